# Initial kernel scaffold; baseline (speedup 1.0000x reference)
import jax
import jax.numpy as jnp
from jax.experimental import pallas as pl


def kernel(x, edge_index, edge_weight, pool_index, emb_matrix, gamma, beta, running_mean, running_var, W, b):
    raise NotImplementedError("write your pallas kernel here")


# speedup vs baseline: 6.6672x; 6.6672x over previous
import dataclasses
import functools

import jax
import jax.numpy as jnp
from jax import lax
from jax.experimental import pallas as pl
from jax.experimental.pallas import tpu as pltpu
from jax.experimental.pallas import tpu_sc as plsc

N_NODES = 10000
E_EDGES = 320000
F = 128
FH = F // 2
POOL = 1024
NROWS = 640
NC, NS, L = 2, 16, 16

EA = E_EDGES // NS
CA = 400
CD = 80


def _t1_body(x_ref, g_ref, be_ref, mu_ref, var_ref, w_ref, o_ref):
    s = g_ref[...] * lax.rsqrt(var_ref[...] + 1e-5)
    t = be_ref[...] - mu_ref[...] * s
    xbn = x_ref[...] * s + t
    o_ref[...] = jnp.dot(xbn, w_ref[...], preferred_element_type=jnp.float32)


def _t1(x, gamma, beta, mu, var, W):
    n = x.shape[0]
    blk = 1000
    grid = n // blk
    return pl.pallas_call(
        _t1_body,
        grid=(grid,),
        in_specs=[
            pl.BlockSpec((blk, F), lambda i: (i, 0)),
            pl.BlockSpec((1, F), lambda i: (0, 0)),
            pl.BlockSpec((1, F), lambda i: (0, 0)),
            pl.BlockSpec((1, F), lambda i: (0, 0)),
            pl.BlockSpec((1, F), lambda i: (0, 0)),
            pl.BlockSpec((F, F), lambda i: (0, 0)),
        ],
        out_specs=pl.BlockSpec((blk, F), lambda i: (i, 0)),
        out_shape=jax.ShapeDtypeStruct((n, F), jnp.float32),
    )(x, gamma, beta, mu, var, W)


def _splat(v16, k):
    return lax.gather(
        v16, jnp.full((L, 1), k, jnp.int32),
        lax.GatherDimensionNumbers(offset_dims=(), collapsed_slice_dims=(0,),
                                   start_index_map=(0,)),
        slice_sizes=(1,), mode=lax.GatherScatterMode.PROMISE_IN_BOUNDS)


def _sc_body(src_hbm, dst_hbm, ew_hbm, xwh_hbm, rid_hbm,
             acc_hbm, dinv_hbm,
             deg_ts, dinv_ts, loc40, rows_v, rid_ts,
             srcD, dstD, ewD, dstA, ewA,
             deg_sh, dinv_sh, acc_sh, sem):
    cid = lax.axis_index("c")
    sid = lax.axis_index("s")
    zero16 = jnp.zeros((L,), jnp.float32)

    pltpu.sync_copy(rid_hbm, rid_ts)

    @pl.loop(0, NROWS)
    def _(r):
        deg_ts.at[r][...] = zero16

    @pl.loop(0, CD)
    def _(r):
        for j in range(FH // L):
            rows_v.at[r, pl.ds(j * L, L)][...] = zero16

    @pl.loop(0, 40)
    def _(r):
        loc40.at[r][...] = zero16

    for k in range(640 // CD):
        pltpu.sync_copy(rows_v, acc_sh.at[pl.ds(sid * 640 + k * CD, CD)])
    pltpu.sync_copy(loc40, deg_sh.at[pl.ds(sid * 40, 40)])
    plsc.subcore_barrier()

    baseA = sid * EA

    @pl.loop(0, EA // CA)
    def _(ci):
        off = baseA + ci * CA
        pltpu.sync_copy(dst_hbm.at[pl.ds(off, CA)], dstA)
        pltpu.sync_copy(ew_hbm.at[pl.ds(off, CA)], ewA)

        @pl.loop(0, CA // L)
        def _(j):
            d16 = dstA[pl.ds(j * L, L)]
            w16 = ewA[pl.ds(j * L, L)]
            r16 = jnp.right_shift(d16, 4)
            c16 = jnp.bitwise_and(d16, 15)
            plsc.addupdate_scatter(deg_ts, [r16, c16], w16)

    for k in range(NROWS // 128):
        pltpu.sync_copy(deg_ts.at[pl.ds(k * 128, 128)],
                        deg_sh.at[rid_ts.at[k]], add=True)
    plsc.subcore_barrier()

    r0 = sid * 40
    pltpu.sync_copy(deg_sh.at[pl.ds(r0, 40)], loc40)

    @pl.loop(0, 40)
    def _(r):
        d = loc40.at[r][...] + 1.0
        i = lax.bitcast_convert_type(d, jnp.int32)
        i = jnp.int32(0x5F3759DF) - jnp.right_shift(i, 1)
        y = lax.bitcast_convert_type(i, jnp.float32)
        for _ in range(4):
            y = y * (1.5 - 0.5 * d * y * y)
        loc40.at[r][...] = jnp.where(d > 0.0, y, 0.0)

    pltpu.sync_copy(loc40, dinv_sh.at[pl.ds(r0, 40)])
    plsc.subcore_barrier()
    pltpu.sync_copy(dinv_sh, dinv_ts)

    @pl.when(jnp.logical_and(cid == 0, sid == 0))
    def _():
        pltpu.sync_copy(dinv_sh, dinv_hbm)

    table = xwh_hbm.at[cid]

    @pl.loop(0, EA // CD)
    def _(ci):
        off = baseA + ci * CD
        pltpu.sync_copy(src_hbm.at[pl.ds(off, CD)], srcD)
        pltpu.sync_copy(dst_hbm.at[pl.ds(off, CD)], dstD)
        pltpu.sync_copy(ew_hbm.at[pl.ds(off, CD)], ewD)
        pltpu.async_copy(table.at[srcD], rows_v, sem).wait()

        @pl.loop(0, CD // L)
        def _(j):
            s16 = srcD[pl.ds(j * L, L)]
            d16 = dstD[pl.ds(j * L, L)]
            w16 = ewD[pl.ds(j * L, L)]
            ds_ = plsc.load_gather(
                dinv_ts, [jnp.right_shift(s16, 4), jnp.bitwise_and(s16, 15)])
            dd_ = plsc.load_gather(
                dinv_ts, [jnp.right_shift(d16, 4), jnp.bitwise_and(d16, 15)])
            sc16 = w16 * ds_ * dd_
            for k in range(L):
                e = j * L + k
                w = _splat(sc16, k)
                for f in range(FH // L):
                    rows_v.at[e, pl.ds(f * L, L)][...] = (
                        rows_v.at[e, pl.ds(f * L, L)][...] * w)

        pltpu.sync_copy(rows_v, acc_sh.at[dstD], add=True)

    plsc.subcore_barrier()

    @pl.when(sid < NS - 1)
    def _():
        pltpu.sync_copy(acc_sh.at[pl.ds(sid * 640, 640)],
                        acc_hbm.at[cid].at[pl.ds(sid * 640, 640)])

    @pl.when(sid == NS - 1)
    def _():
        pltpu.sync_copy(acc_sh.at[pl.ds(9600, 400)],
                        acc_hbm.at[cid].at[pl.ds(9600, 400)])


def _sc_edge(src, dst, ew, xwh, rowids):
    mesh = plsc.VectorSubcoreMesh(core_axis_name="c", subcore_axis_name="s")
    cp = pltpu.CompilerParams(needs_layout_passes=False,
                              use_tc_tiling_on_sc=False)
    f = pl.kernel(
        _sc_body,
        mesh=mesh,
        compiler_params=cp,
        out_type=(jax.ShapeDtypeStruct((NC, N_NODES, FH), jnp.float32),
                  jax.ShapeDtypeStruct((NROWS, L), jnp.float32)),
        scratch_types=[
            pltpu.VMEM((NROWS, L), jnp.float32),
            pltpu.VMEM((NROWS, L), jnp.float32),
            pltpu.VMEM((40, L), jnp.float32),
            pltpu.VMEM((CD, FH), jnp.float32),
            pltpu.VMEM((NROWS // 128, 128), jnp.int32),
            pltpu.VMEM((CD,), jnp.int32),
            pltpu.VMEM((CD,), jnp.int32),
            pltpu.VMEM((CD,), jnp.float32),
            pltpu.VMEM((CA,), jnp.int32),
            pltpu.VMEM((CA,), jnp.float32),
            pltpu.VMEM_SHARED((NROWS, L), jnp.float32),
            pltpu.VMEM_SHARED((NROWS, L), jnp.float32),
            pltpu.VMEM_SHARED((NROWS * L, FH), jnp.float32),
            pltpu.SemaphoreType.DMA,
        ],
    )
    return f(src, dst, ew, xwh, rowids)


def _t2_body(p_ref, a0_ref, a1_ref, xw_ref, dinv_ref, b_ref, o_ref):
    @pl.when(pl.program_id(0) == 0)
    def _():
        o_ref[...] = jnp.zeros_like(o_ref)

    dinv = dinv_ref[...]
    acc = jnp.concatenate([a0_ref[...], a1_ref[...]], axis=1)
    x2 = acc + dinv * dinv * xw_ref[...] + b_ref[...]
    o_ref[...] += lax.dot_general(
        p_ref[...], x2, (((0,), (0,)), ((), ())),
        preferred_element_type=jnp.float32)


def _t2(P, acc0, acc1, xw, dinv2d, b2d):
    n = P.shape[0]
    blk = 1000
    grid = n // blk
    return pl.pallas_call(
        _t2_body,
        grid=(grid,),
        in_specs=[
            pl.BlockSpec((blk, POOL), lambda i: (i, 0)),
            pl.BlockSpec((blk, FH), lambda i: (i, 0)),
            pl.BlockSpec((blk, FH), lambda i: (i, 0)),
            pl.BlockSpec((blk, F), lambda i: (i, 0)),
            pl.BlockSpec((blk, 1), lambda i: (i, 0)),
            pl.BlockSpec((1, F), lambda i: (0, 0)),
        ],
        out_specs=pl.BlockSpec((POOL, F), lambda i: (0, 0)),
        out_shape=jax.ShapeDtypeStruct((POOL, F), jnp.float32),
    )(P, acc0, acc1, xw, dinv2d, b2d)


def kernel(x, edge_index, edge_weight, pool_index, emb_matrix,
           gamma, beta, running_mean, running_var, W, b):
    del emb_matrix
    ei = edge_index.astype(jnp.int32)
    src, dst = ei[0], ei[1]
    ew = edge_weight.astype(jnp.float32)

    xw = _t1(x, gamma.reshape(1, F), beta.reshape(1, F),
             running_mean.reshape(1, F), running_var.reshape(1, F), W)
    xwh = jnp.stack([xw[:, :FH], xw[:, FH:]])

    rowids = jnp.arange(NROWS, dtype=jnp.int32).reshape(NROWS // 128, 128)
    acc, dinv16 = _sc_edge(src, dst, ew, xwh, rowids)

    dinv2d = dinv16.reshape(NROWS * L, 1)[:N_NODES]
    return _t2(pool_index, acc[0], acc[1], xw, dinv2d, b.reshape(1, F))
